# baseline (speedup 1.0000x reference)
import functools

import jax
import jax.numpy as jnp
from jax import lax
from jax.experimental import pallas as pl
from jax.experimental.pallas import tpu as pltpu
from jax.experimental.pallas import tpu_sc as plsc

N = 10000
E = 320000
D = 128
LANES = 16
NC = 2
NS = 16
NW = NC * NS
EPT = E // NW
CHUNK = 40
NCHUNK = EPT // CHUNK
NGB = 4
NSB = 2
PREF = 2
NROUND = (NCHUNK - NSB) // NGB
TAIL = NCHUNK - NROUND * NGB
RPT = 632
RPT_LAST = N - (NS - 1) * RPT


def _sc_aggregate(x, col1d, row1d, w1d):
    mesh = plsc.VectorSubcoreMesh(core_axis_name="c", subcore_axis_name="s")

    @functools.partial(
        pl.kernel,
        mesh=mesh,
        out_type=jax.ShapeDtypeStruct((NC * N, D), jnp.float32),
        scratch_types=(
            [
                pltpu.VMEM_SHARED((N, D), jnp.float32),
                pltpu.VMEM((EPT,), jnp.int32),
            ]
            + [pltpu.VMEM((CHUNK, D), jnp.float32)] * NGB
            + [pltpu.VMEM((CHUNK, D), jnp.float32)] * NSB
            + [pltpu.VMEM((CHUNK,), jnp.int32)] * NSB
            + [pltpu.VMEM((CHUNK,), jnp.float32)] * NGB
            + [pltpu.SemaphoreType.DMA] * NGB
            + [pltpu.SemaphoreType.DMA] * NSB
            + [pltpu.SemaphoreType.DMA] * NSB
            + [pltpu.SemaphoreType.DMA] * NGB
        ),
    )
    def k(x_hbm, col_hbm, row_hbm, w_hbm, out_hbm, acc, col_v, *ring):
        o = 0
        gbuf = ring[o:o + NGB]; o += NGB
        sbuf = ring[o:o + NSB]; o += NSB
        ridx = ring[o:o + NSB]; o += NSB
        wbuf = ring[o:o + NGB]; o += NGB
        gsem = ring[o:o + NGB]; o += NGB
        ssem = ring[o:o + NSB]; o += NSB
        risem = ring[o:o + NSB]; o += NSB
        wsem = ring[o:o + NGB]; o += NGB
        cid = lax.axis_index("c")
        sid = lax.axis_index("s")
        wid = cid * NS + sid
        ebase = wid * EPT

        pltpu.sync_copy(col_hbm.at[pl.ds(ebase, EPT)], col_v)

        zeros = jnp.zeros((LANES,), jnp.float32)
        zb = sbuf[0]

        def zbody(j, c_):
            for c in range(D // LANES):
                zb[j, pl.ds(c * LANES, LANES)] = zeros
            return c_

        lax.fori_loop(0, CHUNK, zbody, 0)
        r0 = sid * RPT

        def zero_rows(base, nrows):
            for i in range(nrows // CHUNK):
                pltpu.sync_copy(zb, acc.at[pl.ds(base + i * CHUNK, CHUNK)])
            rem = nrows % CHUNK
            if rem:
                pltpu.sync_copy(zb.at[pl.ds(0, rem)],
                                acc.at[pl.ds(base + (nrows // CHUNK) * CHUNK,
                                             rem)])

        zero_rows(r0, RPT_LAST)

        @pl.when(sid < NS - 1)
        def _():
            zero_rows(r0 + RPT_LAST, RPT - RPT_LAST)

        plsc.subcore_barrier()

        def gather_start(g, b):
            pltpu.make_async_copy(
                x_hbm.at[col_v.at[pl.ds(g * CHUNK, CHUNK)]], gbuf[b],
                gsem[b]).start()

        def gather_wait(g, b):
            pltpu.make_async_copy(
                x_hbm.at[col_v.at[pl.ds(g * CHUNK, CHUNK)]], gbuf[b],
                gsem[b]).wait()

        def ridx_start(g, p):
            pltpu.make_async_copy(
                row_hbm.at[pl.ds(ebase + g * CHUNK, CHUNK)], ridx[p],
                risem[p]).start()

        def ridx_wait(g, p):
            pltpu.make_async_copy(
                row_hbm.at[pl.ds(ebase + g * CHUNK, CHUNK)], ridx[p],
                risem[p]).wait()

        def wbuf_start(g, b):
            pltpu.make_async_copy(
                w_hbm.at[pl.ds(ebase + g * CHUNK, CHUNK)], wbuf[b],
                wsem[b]).start()

        def wbuf_wait(g, b):
            pltpu.make_async_copy(
                w_hbm.at[pl.ds(ebase + g * CHUNK, CHUNK)], wbuf[b],
                wsem[b]).wait()

        def scatter_wait(p):
            pltpu.make_async_copy(sbuf[p], acc.at[ridx[p]], ssem[p]).wait()

        def do_chunk(g, b, p):
            gather_wait(g, b)

            @pl.when(g >= NSB)
            def _():
                scatter_wait(p)

            ridx_start(g, p)
            wbuf_wait(g, b)
            src, dst, wv = gbuf[b], sbuf[p], wbuf[b]
            for jj in range(CHUNK // LANES):
                w16 = wv[pl.ds(jj * LANES, LANES)]
                for l in range(LANES):
                    j = jj * LANES + l
                    wsplat = jnp.broadcast_to(w16[l], (LANES,))
                    for c in range(D // LANES):
                        sl = pl.ds(c * LANES, LANES)
                        dst[j, sl] = src[j, sl] * wsplat
            if CHUNK % LANES:
                wtail = wv[pl.ds(CHUNK - LANES, LANES)]
                for j in range((CHUNK // LANES) * LANES, CHUNK):
                    l = j - (CHUNK - LANES)
                    wsplat = jnp.broadcast_to(wtail[l], (LANES,))
                    for c in range(D // LANES):
                        sl = pl.ds(c * LANES, LANES)
                        dst[j, sl] = src[j, sl] * wsplat
            ridx_wait(g, p)
            pltpu.async_copy(dst, acc.at[ridx[p]], ssem[p], add=True)

        for g0 in range(NGB):
            wbuf_start(g0, g0)
        for g0 in range(PREF):
            gather_start(g0, g0)

        def round_body(q, c_):
            for b in range(NGB):
                g = q * NGB + b
                p = b % NSB
                do_chunk(g, b, p)
                gather_start(g + PREF, (b + PREF) % NGB)

                @pl.when(g + NGB < NCHUNK)
                def _(b=b):
                    wbuf_start(g + NGB, b)

            return c_

        lax.fori_loop(0, NROUND, round_body, 0)

        for t in range(TAIL):
            g = NROUND * NGB + t
            do_chunk(g, t, t % NSB)
        for t in range(TAIL):
            scatter_wait((NROUND * NGB + t) % NSB)

        plsc.subcore_barrier()

        @pl.when(sid < NS - 1)
        def _():
            pltpu.sync_copy(acc.at[pl.ds(r0, RPT)],
                            out_hbm.at[pl.ds(cid * N + r0, RPT)])

        @pl.when(sid == NS - 1)
        def _():
            pltpu.sync_copy(acc.at[pl.ds(r0, RPT_LAST)],
                            out_hbm.at[pl.ds(cid * N + r0, RPT_LAST)])

    return k(x, col1d, row1d, w1d)


def _tc_body(x_ref, p0_ref, p1_ref, ws_ref, wn_ref, b_ref, o_ref):
    dn = (((1,), (1,)), ((), ()))
    o_ref[...] = (
        lax.dot_general(x_ref[...], ws_ref[...], dn,
                        preferred_element_type=jnp.float32)
        + b_ref[...]
        + lax.dot_general(p0_ref[...] + p1_ref[...], wn_ref[...], dn,
                          preferred_element_type=jnp.float32)
    )


def _tc_combine(x, partial, W_self, W_neigh, b2d):
    BM = 1000
    nblk = N // BM
    return pl.pallas_call(
        _tc_body,
        grid=(nblk,),
        in_specs=[
            pl.BlockSpec((BM, D), lambda i: (i, 0)),
            pl.BlockSpec((BM, D), lambda i: (i, 0)),
            pl.BlockSpec((BM, D), lambda i, _n=nblk: (i + _n, 0)),
            pl.BlockSpec((D, D), lambda i: (0, 0)),
            pl.BlockSpec((D, D), lambda i: (0, 0)),
            pl.BlockSpec((1, D), lambda i: (0, 0)),
        ],
        out_specs=pl.BlockSpec((BM, D), lambda i: (i, 0)),
        out_shape=jax.ShapeDtypeStruct((N, D), jnp.float32),
    )(x, partial, partial, W_self, W_neigh, b2d)


def kernel(x, edge_index, edge_weight, W_self, b_self, W_neigh):
    row1d = edge_index[0].astype(jnp.int32)
    col1d = edge_index[1].astype(jnp.int32)
    w1d = edge_weight.astype(jnp.float32)
    partial = _sc_aggregate(x, col1d, row1d, w1d)
    return _tc_combine(x, partial, W_self, W_neigh, b_self.reshape(1, D))


# speedup vs baseline: 1.0173x; 1.0173x over previous
import functools

import jax
import jax.numpy as jnp
import numpy as np
from jax import lax
from jax.experimental import pallas as pl
from jax.experimental.pallas import tpu as pltpu
from jax.experimental.pallas import tpu_sc as plsc

N = 10000
E = 320000
D = 128
LANES = 16
NC = 2
NS = 16
NW = NC * NS
EPT = E // NW
CHUNK = 40
NCHUNK = EPT // CHUNK
NBUF = 5
PREF = 3
NROUND = NCHUNK // NBUF
TAIL = NCHUNK - NROUND * NBUF
_PB = np.stack([np.arange(16), np.arange(16) + 16], axis=1).reshape(32)
PERM = np.concatenate([_PB + 32 * i for i in range(D // 32)])
RPT = 632
RPT_LAST = N - (NS - 1) * RPT


def _sc_aggregate(x, col1d, row1d, w1d):
    mesh = plsc.VectorSubcoreMesh(core_axis_name="c", subcore_axis_name="s")

    @functools.partial(
        pl.kernel,
        mesh=mesh,
        compiler_params=pltpu.CompilerParams(use_tc_tiling_on_sc=False),
        out_type=jax.ShapeDtypeStruct((NC * N, D), jnp.float32),
        scratch_types=(
            [
                pltpu.VMEM_SHARED((N, D), jnp.float32),
                pltpu.VMEM((EPT,), jnp.int32),
            ]
            + [pltpu.VMEM((CHUNK, D // 2), jnp.int32)] * NBUF
            + [pltpu.VMEM((CHUNK, D), jnp.float32)] * NBUF
            + [pltpu.VMEM((CHUNK,), jnp.int32)] * NBUF
            + [pltpu.VMEM((CHUNK,), jnp.float32)] * NBUF
            + [pltpu.SemaphoreType.DMA] * (4 * NBUF)
        ),
    )
    def k(x_hbm, col_hbm, row_hbm, w_hbm, out_hbm,
          acc, col_v, *ring):
        bufs = ring[0:NBUF]
        sbufs = ring[NBUF:2 * NBUF]
        ridxs = ring[2 * NBUF:3 * NBUF]
        wbufs = ring[3 * NBUF:4 * NBUF]
        gsems = ring[4 * NBUF:5 * NBUF]
        ssems = ring[5 * NBUF:6 * NBUF]
        risems = ring[6 * NBUF:7 * NBUF]
        wsems = ring[7 * NBUF:8 * NBUF]
        buf0 = sbufs[0]
        cid = lax.axis_index("c")
        sid = lax.axis_index("s")
        wid = cid * NS + sid

        pltpu.sync_copy(col_hbm.at[pl.ds(wid * EPT, EPT)], col_v)

        zeros = jnp.zeros((LANES,), jnp.float32)

        def zbody(j, c_):
            for c in range(D // LANES):
                buf0[j, pl.ds(c * LANES, LANES)] = zeros
            return c_

        lax.fori_loop(0, CHUNK, zbody, 0)
        r0 = sid * RPT

        def zero_rows(base, nrows):
            for i in range(nrows // CHUNK):
                pltpu.sync_copy(buf0, acc.at[pl.ds(base + i * CHUNK, CHUNK)])
            rem = nrows % CHUNK
            if rem:
                pltpu.sync_copy(buf0.at[pl.ds(0, rem)],
                                acc.at[pl.ds(base + (nrows // CHUNK) * CHUNK,
                                             rem)])

        zero_rows(r0, RPT_LAST)

        @pl.when(sid < NS - 1)
        def _():
            zero_rows(r0 + RPT_LAST, RPT - RPT_LAST)

        plsc.subcore_barrier()

        def gather_start(g, buf, sem):
            pltpu.make_async_copy(
                x_hbm.at[col_v.at[pl.ds(g * CHUNK, CHUNK)]], buf, sem).start()

        def gather_wait(g, buf, sem):
            pltpu.make_async_copy(
                x_hbm.at[col_v.at[pl.ds(g * CHUNK, CHUNK)]], buf, sem).wait()

        def ridx_start(g, ridx, sem):
            pltpu.make_async_copy(
                row_hbm.at[pl.ds(wid * EPT + g * CHUNK, CHUNK)], ridx,
                sem).start()

        def ridx_wait(g, ridx, sem):
            pltpu.make_async_copy(
                row_hbm.at[pl.ds(wid * EPT + g * CHUNK, CHUNK)], ridx,
                sem).wait()

        def wbuf_start(g, wbuf, sem):
            pltpu.make_async_copy(
                w_hbm.at[pl.ds(wid * EPT + g * CHUNK, CHUNK)], wbuf,
                sem).start()

        def wbuf_wait(g, wbuf, sem):
            pltpu.make_async_copy(
                w_hbm.at[pl.ds(wid * EPT + g * CHUNK, CHUNK)], wbuf,
                sem).wait()

        def do_chunk(g, buf, sbuf, ridx, wv, ssem, risem, wsem):
            wbuf_wait(g, wv, wsem)
            for jj in range(CHUNK // LANES):
                w16 = wv[pl.ds(jj * LANES, LANES)]
                for l in range(LANES):
                    j = jj * LANES + l
                    wsplat = jnp.broadcast_to(w16[l], (LANES,))
                    for c2 in range(D // (2 * LANES)):
                        v = buf[j, pl.ds(c2 * LANES, LANES)]
                        lo = jax.lax.bitcast_convert_type(
                            v << 16, jnp.float32)
                        hi = jax.lax.bitcast_convert_type(
                            v & jnp.int32(-65536), jnp.float32)
                        sl0 = pl.ds(c2 * 2 * LANES, LANES)
                        sl1 = pl.ds(c2 * 2 * LANES + LANES, LANES)
                        sbuf[j, sl0] = lo * wsplat
                        sbuf[j, sl1] = hi * wsplat
            if CHUNK % LANES:
                wtail = wv[pl.ds(CHUNK - LANES, LANES)]
                for j in range((CHUNK // LANES) * LANES, CHUNK):
                    l = j - (CHUNK - LANES)
                    wsplat = jnp.broadcast_to(wtail[l], (LANES,))
                    for c2 in range(D // (2 * LANES)):
                        v = buf[j, pl.ds(c2 * LANES, LANES)]
                        lo = jax.lax.bitcast_convert_type(
                            v << 16, jnp.float32)
                        hi = jax.lax.bitcast_convert_type(
                            v & jnp.int32(-65536), jnp.float32)
                        sl0 = pl.ds(c2 * 2 * LANES, LANES)
                        sl1 = pl.ds(c2 * 2 * LANES + LANES, LANES)
                        sbuf[j, sl0] = lo * wsplat
                        sbuf[j, sl1] = hi * wsplat
            ridx_wait(g, ridx, risem)
            pltpu.async_copy(sbuf, acc.at[ridx], ssem, add=True)

        def scatter_wait(sbuf, ridx, ssem):
            pltpu.make_async_copy(sbuf, acc.at[ridx], ssem).wait()

        for b in range(PREF):
            ridx_start(b, ridxs[b], risems[b])
            wbuf_start(b, wbufs[b], wsems[b])
            gather_start(b, bufs[b], gsems[b])

        def round_body(q, c_):
            for b in range(NBUF):
                g = q * NBUF + b
                gather_wait(g, bufs[b], gsems[b])
                do_chunk(g, bufs[b], sbufs[b], ridxs[b], wbufs[b],
                         ssems[b], risems[b], wsems[b])
                bb = (b + PREF) % NBUF

                @pl.when(g >= NBUF - PREF)
                def _(bb=bb):
                    scatter_wait(sbufs[bb], ridxs[bb], ssems[bb])

                @pl.when(g + PREF < NCHUNK)
                def _(bb=bb):
                    ridx_start(g + PREF, ridxs[bb], risems[bb])
                    wbuf_start(g + PREF, wbufs[bb], wsems[bb])
                    gather_start(g + PREF, bufs[bb], gsems[bb])

            return c_

        lax.fori_loop(0, NROUND, round_body, 0)

        for gl in range(NCHUNK - (NBUF - PREF), NCHUNK):
            b = gl % NBUF
            scatter_wait(sbufs[b], ridxs[b], ssems[b])

        plsc.subcore_barrier()

        @pl.when(sid < NS - 1)
        def _():
            pltpu.sync_copy(acc.at[pl.ds(r0, RPT)],
                            out_hbm.at[pl.ds(cid * N + r0, RPT)])

        @pl.when(sid == NS - 1)
        def _():
            pltpu.sync_copy(acc.at[pl.ds(r0, RPT_LAST)],
                            out_hbm.at[pl.ds(cid * N + r0, RPT_LAST)])

    return k(x, col1d, row1d, w1d)


def _tc_body(x_ref, p0_ref, p1_ref, ws_ref, wn_ref, b_ref, o_ref):
    dn = (((1,), (1,)), ((), ()))
    o_ref[...] = (
        lax.dot_general(x_ref[...], ws_ref[...], dn,
                        preferred_element_type=jnp.float32)
        + b_ref[...]
        + lax.dot_general(p0_ref[...] + p1_ref[...], wn_ref[...], dn,
                          preferred_element_type=jnp.float32)
    )


def _tc_combine(x, partial, W_self, W_neigh, b2d):
    BM = 1000
    nblk = N // BM
    return pl.pallas_call(
        _tc_body,
        grid=(nblk,),
        in_specs=[
            pl.BlockSpec((BM, D), lambda i: (i, 0)),
            pl.BlockSpec((BM, D), lambda i: (i, 0)),
            pl.BlockSpec((BM, D), lambda i, _n=nblk: (i + _n, 0)),
            pl.BlockSpec((D, D), lambda i: (0, 0)),
            pl.BlockSpec((D, D), lambda i: (0, 0)),
            pl.BlockSpec((1, D), lambda i: (0, 0)),
        ],
        out_specs=pl.BlockSpec((BM, D), lambda i: (i, 0)),
        out_shape=jax.ShapeDtypeStruct((N, D), jnp.float32),
    )(x, partial, partial, W_self, W_neigh, b2d)


def kernel(x, edge_index, edge_weight, W_self, b_self, W_neigh):
    row1d = edge_index[0].astype(jnp.int32)
    col1d = edge_index[1].astype(jnp.int32)
    w1d = edge_weight.astype(jnp.float32)
    xb = x[:, PERM].astype(jnp.bfloat16)
    xi = jax.lax.bitcast_convert_type(xb.reshape(N, D // 2, 2), jnp.int32)
    partial = _sc_aggregate(xi, col1d, row1d, w1d)
    return _tc_combine(x, partial, W_self, W_neigh, b_self.reshape(1, D))


# speedup vs baseline: 1.1718x; 1.1519x over previous
import functools

import jax
import jax.numpy as jnp
from jax import lax
from jax.experimental import pallas as pl
from jax.experimental.pallas import tpu as pltpu
from jax.experimental.pallas import tpu_sc as plsc

N = 10000
E = 320000
D = 128
LANES = 16
NC = 2
NS = 16
NW = NC * NS
EPT = E // NW
CHUNK = 40
NCHUNK = EPT // CHUNK
NBUF = 5
PREF = 3
NROUND = NCHUNK // NBUF
TAIL = NCHUNK - NROUND * NBUF
GRP = 8
RPT = 632
RPT_LAST = N - (NS - 1) * RPT


def _sc_aggregate(x, ei1d, w1d):
    mesh = plsc.VectorSubcoreMesh(core_axis_name="c", subcore_axis_name="s")

    @functools.partial(
        pl.kernel,
        mesh=mesh,
        out_type=jax.ShapeDtypeStruct((NC * N, D), jnp.float32),
        scratch_types=(
            [
                pltpu.VMEM_SHARED((N, D), jnp.float32),
                pltpu.VMEM((EPT,), jnp.int32),
                pltpu.VMEM((EPT + LANES,), jnp.float32),
            ]
            + [pltpu.VMEM((CHUNK, D), jnp.float32)] * NBUF
            + [pltpu.VMEM((CHUNK,), jnp.int32)] * NBUF
            + [pltpu.SemaphoreType.DMA] * (3 * NBUF)
        ),
    )
    def k(x_hbm, ei_hbm, w_hbm, out_hbm,
          acc, col_v, w_v, *ring):
        bufs = ring[0:NBUF]
        ridxs = ring[NBUF:2 * NBUF]
        gsems = ring[2 * NBUF:3 * NBUF]
        ssems = ring[3 * NBUF:4 * NBUF]
        risems = ring[4 * NBUF:5 * NBUF]
        buf0 = bufs[0]
        cid = lax.axis_index("c")
        sid = lax.axis_index("s")
        wid = cid * NS + sid

        pltpu.sync_copy(ei_hbm.at[pl.ds(E + wid * EPT, EPT)], col_v)
        pltpu.sync_copy(w_hbm.at[pl.ds(wid * EPT, EPT)], w_v.at[pl.ds(0, EPT)])

        zeros = jnp.zeros((LANES,), jnp.float32)

        def zbody(j, c_):
            for c in range(D // LANES):
                buf0[j, pl.ds(c * LANES, LANES)] = zeros
            return c_

        lax.fori_loop(0, CHUNK, zbody, 0)
        r0 = sid * RPT

        def zero_rows(base, nrows):
            for i in range(nrows // CHUNK):
                pltpu.sync_copy(buf0, acc.at[pl.ds(base + i * CHUNK, CHUNK)])
            rem = nrows % CHUNK
            if rem:
                pltpu.sync_copy(buf0.at[pl.ds(0, rem)],
                                acc.at[pl.ds(base + (nrows // CHUNK) * CHUNK,
                                             rem)])

        zero_rows(r0, RPT_LAST)

        @pl.when(sid < NS - 1)
        def _():
            zero_rows(r0 + RPT_LAST, RPT - RPT_LAST)

        plsc.subcore_barrier()

        def gather_start(g, buf, sem):
            pltpu.make_async_copy(
                x_hbm.at[col_v.at[pl.ds(g * CHUNK, CHUNK)]], buf, sem).start()

        def gather_wait(g, buf, sem):
            pltpu.make_async_copy(
                x_hbm.at[col_v.at[pl.ds(g * CHUNK, CHUNK)]], buf, sem).wait()

        def ridx_start(g, ridx, sem):
            pltpu.make_async_copy(
                ei_hbm.at[pl.ds(wid * EPT + g * CHUNK, CHUNK)], ridx,
                sem).start()

        def ridx_wait(g, ridx, sem):
            pltpu.make_async_copy(
                ei_hbm.at[pl.ds(wid * EPT + g * CHUNK, CHUNK)], ridx,
                sem).wait()

        def do_chunk(g, buf, ridx, ssem, risem):
            wbase = g * CHUNK
            for jj in range(CHUNK // LANES):
                w16 = w_v[pl.ds(wbase + jj * LANES, LANES)]
                for l in range(LANES):
                    j = jj * LANES + l
                    wsplat = jnp.broadcast_to(w16[l], (LANES,))
                    for c in range(D // LANES):
                        sl = pl.ds(c * LANES, LANES)
                        buf[j, sl] = buf[j, sl] * wsplat
            if CHUNK % LANES:
                wtail = w_v[pl.ds(wbase + CHUNK - LANES, LANES)]
                for j in range((CHUNK // LANES) * LANES, CHUNK):
                    l = j - (CHUNK - LANES)
                    wsplat = jnp.broadcast_to(wtail[l], (LANES,))
                    for c in range(D // LANES):
                        sl = pl.ds(c * LANES, LANES)
                        buf[j, sl] = buf[j, sl] * wsplat
            ridx_wait(g, ridx, risem)
            pltpu.async_copy(buf, acc.at[ridx], ssem, add=True)

        def scatter_wait(buf, ridx, ssem):
            pltpu.make_async_copy(buf, acc.at[ridx], ssem).wait()

        for b in range(PREF):
            ridx_start(b, ridxs[b], risems[b])
            gather_start(b, bufs[b], gsems[b])

        def round_body(q, c_):
            for b in range(NBUF):
                g = q * NBUF + b
                gather_wait(g, bufs[b], gsems[b])
                do_chunk(g, bufs[b], ridxs[b], ssems[b], risems[b])
                bb = (b + PREF) % NBUF

                @pl.when(g >= NBUF - PREF)
                def _(bb=bb):
                    scatter_wait(bufs[bb], ridxs[bb], ssems[bb])

                @pl.when(g + PREF < NCHUNK)
                def _(bb=bb):
                    ridx_start(g + PREF, ridxs[bb], risems[bb])
                    gather_start(g + PREF, bufs[bb], gsems[bb])

            return c_

        lax.fori_loop(0, NROUND, round_body, 0)

        for gl in range(NCHUNK - (NBUF - PREF), NCHUNK):
            b = gl % NBUF
            scatter_wait(bufs[b], ridxs[b], ssems[b])

        plsc.subcore_barrier()

        @pl.when(sid < NS - 1)
        def _():
            pltpu.sync_copy(acc.at[pl.ds(r0, RPT)],
                            out_hbm.at[pl.ds(cid * N + r0, RPT)])

        @pl.when(sid == NS - 1)
        def _():
            pltpu.sync_copy(acc.at[pl.ds(r0, RPT_LAST)],
                            out_hbm.at[pl.ds(cid * N + r0, RPT_LAST)])

    return k(x, ei1d, w1d)


def _tc_body(x_ref, p0_ref, p1_ref, ws_ref, wn_ref, b_ref, o_ref):
    dn = (((1,), (1,)), ((), ()))
    o_ref[...] = (
        lax.dot_general(x_ref[...], ws_ref[...], dn,
                        preferred_element_type=jnp.float32)
        + b_ref[...]
        + lax.dot_general(p0_ref[...] + p1_ref[...], wn_ref[...], dn,
                          preferred_element_type=jnp.float32)
    )


def _tc_combine(x, partial, W_self, W_neigh, b2d):
    BM = 1000
    nblk = N // BM
    return pl.pallas_call(
        _tc_body,
        grid=(nblk,),
        in_specs=[
            pl.BlockSpec((BM, D), lambda i: (i, 0)),
            pl.BlockSpec((BM, D), lambda i: (i, 0)),
            pl.BlockSpec((BM, D), lambda i, _n=nblk: (i + _n, 0)),
            pl.BlockSpec((D, D), lambda i: (0, 0)),
            pl.BlockSpec((D, D), lambda i: (0, 0)),
            pl.BlockSpec((1, D), lambda i: (0, 0)),
        ],
        out_specs=pl.BlockSpec((BM, D), lambda i: (i, 0)),
        out_shape=jax.ShapeDtypeStruct((N, D), jnp.float32),
    )(x, partial, partial, W_self, W_neigh, b2d)


def kernel(x, edge_index, edge_weight, W_self, b_self, W_neigh):
    ei1d = edge_index.astype(jnp.int32).reshape(2 * E)
    w1d = edge_weight.astype(jnp.float32)
    partial = _sc_aggregate(x, ei1d, w1d)
    return _tc_combine(x, partial, W_self, W_neigh, b_self.reshape(1, D))


# speedup vs baseline: 1.1763x; 1.0038x over previous
import functools

import jax
import jax.numpy as jnp
from jax import lax
from jax.experimental import pallas as pl
from jax.experimental.pallas import tpu as pltpu
from jax.experimental.pallas import tpu_sc as plsc

N = 10000
E = 320000
D = 128
LANES = 16
NC = 2
NS = 16
NW = NC * NS
EPT = E // NW
CHUNK = 40
NCHUNK = EPT // CHUNK
NBUF = 5
PREF = 3
NROUND = NCHUNK // NBUF
TAIL = NCHUNK - NROUND * NBUF
GRP = 8
RPT = 632
RPT_LAST = N - (NS - 1) * RPT


def _sc_aggregate(x, ei1d, w1d):
    mesh = plsc.VectorSubcoreMesh(core_axis_name="c", subcore_axis_name="s")

    @functools.partial(
        pl.kernel,
        mesh=mesh,
        out_type=jax.ShapeDtypeStruct((NC * N, D), jnp.float32),
        scratch_types=(
            [
                pltpu.VMEM_SHARED((N, D), jnp.float32),
                pltpu.VMEM((EPT,), jnp.int32),
                pltpu.VMEM((EPT + LANES,), jnp.float32),
            ]
            + [pltpu.VMEM((CHUNK, D), jnp.float32)] * NBUF
            + [pltpu.VMEM((CHUNK,), jnp.int32)] * NBUF
            + [pltpu.SemaphoreType.DMA] * (3 * NBUF)
        ),
    )
    def k(x_hbm, ei_hbm, w_hbm, out_hbm,
          acc, col_v, w_v, *ring):
        bufs = ring[0:NBUF]
        ridxs = ring[NBUF:2 * NBUF]
        gsems = ring[2 * NBUF:3 * NBUF]
        ssems = ring[3 * NBUF:4 * NBUF]
        risems = ring[4 * NBUF:5 * NBUF]
        buf0 = bufs[0]
        cid = lax.axis_index("c")
        sid = lax.axis_index("s")
        wid = cid * NS + sid

        pltpu.sync_copy(ei_hbm.at[pl.ds(E + wid * EPT, EPT)], col_v)
        pltpu.sync_copy(w_hbm.at[pl.ds(wid * EPT, EPT)], w_v.at[pl.ds(0, EPT)])

        zeros = jnp.zeros((LANES,), jnp.float32)

        def zbody(j, c_):
            for c in range(D // LANES):
                buf0[j, pl.ds(c * LANES, LANES)] = zeros
            return c_

        lax.fori_loop(0, CHUNK, zbody, 0)
        r0 = sid * RPT

        def zero_rows(base, nrows):
            for i in range(nrows // CHUNK):
                pltpu.sync_copy(buf0, acc.at[pl.ds(base + i * CHUNK, CHUNK)])
            rem = nrows % CHUNK
            if rem:
                pltpu.sync_copy(buf0.at[pl.ds(0, rem)],
                                acc.at[pl.ds(base + (nrows // CHUNK) * CHUNK,
                                             rem)])

        zero_rows(r0, RPT_LAST)

        @pl.when(sid < NS - 1)
        def _():
            zero_rows(r0 + RPT_LAST, RPT - RPT_LAST)

        plsc.subcore_barrier()

        def gather_start(g, buf, sem):
            pltpu.make_async_copy(
                x_hbm.at[col_v.at[pl.ds(g * CHUNK, CHUNK)]], buf, sem).start()

        def gather_wait(g, buf, sem):
            pltpu.make_async_copy(
                x_hbm.at[col_v.at[pl.ds(g * CHUNK, CHUNK)]], buf, sem).wait()

        def ridx_start(g, ridx, sem):
            pltpu.make_async_copy(
                ei_hbm.at[pl.ds(wid * EPT + g * CHUNK, CHUNK)], ridx,
                sem).start()

        def ridx_wait(g, ridx, sem):
            pltpu.make_async_copy(
                ei_hbm.at[pl.ds(wid * EPT + g * CHUNK, CHUNK)], ridx,
                sem).wait()

        def do_chunk(g, buf, ridx, ssem, risem):
            wbase = g * CHUNK
            for jj in range(CHUNK // LANES):
                w16 = w_v[pl.ds(wbase + jj * LANES, LANES)]
                for l in range(LANES):
                    j = jj * LANES + l
                    wsplat = jnp.broadcast_to(w16[l], (LANES,))
                    for c in range(D // LANES):
                        sl = pl.ds(c * LANES, LANES)
                        buf[j, sl] = buf[j, sl] * wsplat
            if CHUNK % LANES:
                wtail = w_v[pl.ds(wbase + CHUNK - LANES, LANES)]
                for j in range((CHUNK // LANES) * LANES, CHUNK):
                    l = j - (CHUNK - LANES)
                    wsplat = jnp.broadcast_to(wtail[l], (LANES,))
                    for c in range(D // LANES):
                        sl = pl.ds(c * LANES, LANES)
                        buf[j, sl] = buf[j, sl] * wsplat
            ridx_wait(g, ridx, risem)
            pltpu.async_copy(buf, acc.at[ridx], ssem, add=True)

        def scatter_wait(buf, ridx, ssem):
            pltpu.make_async_copy(buf, acc.at[ridx], ssem).wait()

        for b in range(PREF):
            ridx_start(b, ridxs[b], risems[b])
            gather_start(b, bufs[b], gsems[b])

        def round_body(q, c_):
            for b in range(NBUF):
                g = q * NBUF + b
                gather_wait(g, bufs[b], gsems[b])
                do_chunk(g, bufs[b], ridxs[b], ssems[b], risems[b])
                bb = (b + PREF) % NBUF

                @pl.when(g >= NBUF - PREF)
                def _(bb=bb):
                    scatter_wait(bufs[bb], ridxs[bb], ssems[bb])

                @pl.when(g + PREF < NCHUNK)
                def _(bb=bb):
                    ridx_start(g + PREF, ridxs[bb], risems[bb])
                    gather_start(g + PREF, bufs[bb], gsems[bb])

            return c_

        lax.fori_loop(0, NROUND, round_body, 0)

        for gl in range(NCHUNK - (NBUF - PREF), NCHUNK):
            b = gl % NBUF
            scatter_wait(bufs[b], ridxs[b], ssems[b])

        plsc.subcore_barrier()

        @pl.when(sid < NS - 1)
        def _():
            pltpu.sync_copy(acc.at[pl.ds(r0, RPT)],
                            out_hbm.at[pl.ds(cid * N + r0, RPT)])

        @pl.when(sid == NS - 1)
        def _():
            pltpu.sync_copy(acc.at[pl.ds(r0, RPT_LAST)],
                            out_hbm.at[pl.ds(cid * N + r0, RPT_LAST)])

    return k(x, ei1d, w1d)


BM = 1000
_NBLK = N // BM


def _tc_dense_body(x_ref, ws_ref, b_ref, o_ref):
    dn = (((1,), (1,)), ((), ()))
    o_ref[...] = lax.dot_general(
        x_ref[...], ws_ref[...], dn,
        preferred_element_type=jnp.float32) + b_ref[...]


def _tc_dense(x, W_self, b2d):
    return pl.pallas_call(
        _tc_dense_body,
        grid=(_NBLK,),
        in_specs=[
            pl.BlockSpec((BM, D), lambda i: (i, 0)),
            pl.BlockSpec((D, D), lambda i: (0, 0)),
            pl.BlockSpec((1, D), lambda i: (0, 0)),
        ],
        out_specs=pl.BlockSpec((BM, D), lambda i: (i, 0)),
        out_shape=jax.ShapeDtypeStruct((N, D), jnp.float32),
    )(x, W_self, b2d)


def _tc_final_body(d_ref, p0_ref, p1_ref, wn_ref, o_ref):
    dn = (((1,), (1,)), ((), ()))
    o_ref[...] = d_ref[...] + lax.dot_general(
        p0_ref[...] + p1_ref[...], wn_ref[...], dn,
        preferred_element_type=jnp.float32)


def _tc_final(dense, partial, W_neigh):
    return pl.pallas_call(
        _tc_final_body,
        grid=(_NBLK,),
        in_specs=[
            pl.BlockSpec((BM, D), lambda i: (i, 0)),
            pl.BlockSpec((BM, D), lambda i: (i, 0)),
            pl.BlockSpec((BM, D), lambda i: (i + _NBLK, 0)),
            pl.BlockSpec((D, D), lambda i: (0, 0)),
        ],
        out_specs=pl.BlockSpec((BM, D), lambda i: (i, 0)),
        out_shape=jax.ShapeDtypeStruct((N, D), jnp.float32),
    )(dense, partial, partial, W_neigh)


def kernel(x, edge_index, edge_weight, W_self, b_self, W_neigh):
    ei1d = edge_index.astype(jnp.int32).reshape(2 * E)
    w1d = edge_weight.astype(jnp.float32)
    partial = _sc_aggregate(x, ei1d, w1d)
    dense = _tc_dense(x, W_self, b_self.reshape(1, D))
    return _tc_final(dense, partial, W_neigh)


# speedup vs baseline: 1.1859x; 1.0082x over previous
import functools

import jax
import jax.numpy as jnp
from jax import lax
from jax.experimental import pallas as pl
from jax.experimental.pallas import tpu as pltpu
from jax.experimental.pallas import tpu_sc as plsc

N = 10000
E = 320000
D = 128
LANES = 16
NC = 2
NS = 16
NW = NC * NS
EPT = E // NW
CHUNK = 40
NCHUNK = EPT // CHUNK
NBUF = 5
PREF = 3
NROUND = NCHUNK // NBUF
TAIL = NCHUNK - NROUND * NBUF
GRP = 8
RPT = 632
RPT_LAST = N - (NS - 1) * RPT


def _sc_aggregate(x, ei1d, w1d):
    mesh = plsc.VectorSubcoreMesh(core_axis_name="c", subcore_axis_name="s")

    @functools.partial(
        pl.kernel,
        mesh=mesh,
        out_type=jax.ShapeDtypeStruct((NC * N, D), jnp.float32),
        scratch_types=(
            [
                pltpu.VMEM_SHARED((N, D), jnp.float32),
                pltpu.VMEM((EPT,), jnp.int32),
                pltpu.VMEM((EPT + LANES,), jnp.float32),
            ]
            + [pltpu.VMEM((CHUNK, D), jnp.float32)] * NBUF
            + [pltpu.VMEM((CHUNK,), jnp.int32)] * NBUF
            + [pltpu.SemaphoreType.DMA] * (3 * NBUF)
        ),
    )
    def k(x_hbm, ei_hbm, w_hbm, out_hbm,
          acc, col_v, w_v, *ring):
        bufs = ring[0:NBUF]
        ridxs = ring[NBUF:2 * NBUF]
        gsems = ring[2 * NBUF:3 * NBUF]
        ssems = ring[3 * NBUF:4 * NBUF]
        risems = ring[4 * NBUF:5 * NBUF]
        cid = lax.axis_index("c")
        sid = lax.axis_index("s")
        wid = cid * NS + sid

        pltpu.sync_copy(ei_hbm.at[pl.ds(E + wid * EPT, EPT)], col_v)
        pltpu.sync_copy(w_hbm.at[pl.ds(wid * EPT, EPT)], w_v.at[pl.ds(0, EPT)])

        def gather_start(g, buf, sem):
            pltpu.make_async_copy(
                x_hbm.at[col_v.at[pl.ds(g * CHUNK, CHUNK)]], buf, sem).start()

        def gather_wait(g, buf, sem):
            pltpu.make_async_copy(
                x_hbm.at[col_v.at[pl.ds(g * CHUNK, CHUNK)]], buf, sem).wait()

        def ridx_start(g, ridx, sem):
            pltpu.make_async_copy(
                ei_hbm.at[pl.ds(wid * EPT + g * CHUNK, CHUNK)], ridx,
                sem).start()

        def ridx_wait(g, ridx, sem):
            pltpu.make_async_copy(
                ei_hbm.at[pl.ds(wid * EPT + g * CHUNK, CHUNK)], ridx,
                sem).wait()

        def do_chunk(g, buf, ridx, ssem, risem):
            wbase = g * CHUNK
            for jj in range(CHUNK // LANES):
                w16 = w_v[pl.ds(wbase + jj * LANES, LANES)]
                for l in range(LANES):
                    j = jj * LANES + l
                    wsplat = jnp.broadcast_to(w16[l], (LANES,))
                    for c in range(D // LANES):
                        sl = pl.ds(c * LANES, LANES)
                        buf[j, sl] = buf[j, sl] * wsplat
            if CHUNK % LANES:
                wtail = w_v[pl.ds(wbase + CHUNK - LANES, LANES)]
                for j in range((CHUNK // LANES) * LANES, CHUNK):
                    l = j - (CHUNK - LANES)
                    wsplat = jnp.broadcast_to(wtail[l], (LANES,))
                    for c in range(D // LANES):
                        sl = pl.ds(c * LANES, LANES)
                        buf[j, sl] = buf[j, sl] * wsplat
            ridx_wait(g, ridx, risem)
            pltpu.async_copy(buf, acc.at[ridx], ssem, add=True)

        def scatter_wait(buf, ridx, ssem):
            pltpu.make_async_copy(buf, acc.at[ridx], ssem).wait()

        for b in range(PREF):
            ridx_start(b, ridxs[b], risems[b])
            gather_start(b, bufs[b], gsems[b])

        zeros = jnp.zeros((LANES,), jnp.float32)
        zb = bufs[NBUF - 1]

        def zbody(j, c_):
            for c in range(D // LANES):
                zb[j, pl.ds(c * LANES, LANES)] = zeros
            return c_

        lax.fori_loop(0, CHUNK, zbody, 0)
        r0 = sid * RPT

        def zero_rows(base, nrows):
            for i in range(nrows // CHUNK):
                pltpu.sync_copy(zb, acc.at[pl.ds(base + i * CHUNK, CHUNK)])
            rem = nrows % CHUNK
            if rem:
                pltpu.sync_copy(zb.at[pl.ds(0, rem)],
                                acc.at[pl.ds(base + (nrows // CHUNK) * CHUNK,
                                             rem)])

        zero_rows(r0, RPT_LAST)

        @pl.when(sid < NS - 1)
        def _():
            zero_rows(r0 + RPT_LAST, RPT - RPT_LAST)

        plsc.subcore_barrier()

        def round_body(q, c_):
            for b in range(NBUF):
                g = q * NBUF + b
                gather_wait(g, bufs[b], gsems[b])
                do_chunk(g, bufs[b], ridxs[b], ssems[b], risems[b])
                bb = (b + PREF) % NBUF

                @pl.when(g >= NBUF - PREF)
                def _(bb=bb):
                    scatter_wait(bufs[bb], ridxs[bb], ssems[bb])

                @pl.when(g + PREF < NCHUNK)
                def _(bb=bb):
                    ridx_start(g + PREF, ridxs[bb], risems[bb])
                    gather_start(g + PREF, bufs[bb], gsems[bb])

            return c_

        lax.fori_loop(0, NROUND, round_body, 0)

        for gl in range(NCHUNK - (NBUF - PREF), NCHUNK):
            b = gl % NBUF
            scatter_wait(bufs[b], ridxs[b], ssems[b])

        plsc.subcore_barrier()

        @pl.when(sid < NS - 1)
        def _():
            pltpu.sync_copy(acc.at[pl.ds(r0, RPT)],
                            out_hbm.at[pl.ds(cid * N + r0, RPT)])

        @pl.when(sid == NS - 1)
        def _():
            pltpu.sync_copy(acc.at[pl.ds(r0, RPT_LAST)],
                            out_hbm.at[pl.ds(cid * N + r0, RPT_LAST)])

    return k(x, ei1d, w1d)


BM = 1000
_NBLK = N // BM


def _tc_dense_body(x_ref, ws_ref, b_ref, o_ref):
    dn = (((1,), (1,)), ((), ()))
    o_ref[...] = lax.dot_general(
        x_ref[...], ws_ref[...], dn,
        preferred_element_type=jnp.float32) + b_ref[...]


def _tc_dense(x, W_self, b2d):
    return pl.pallas_call(
        _tc_dense_body,
        grid=(_NBLK,),
        in_specs=[
            pl.BlockSpec((BM, D), lambda i: (i, 0)),
            pl.BlockSpec((D, D), lambda i: (0, 0)),
            pl.BlockSpec((1, D), lambda i: (0, 0)),
        ],
        out_specs=pl.BlockSpec((BM, D), lambda i: (i, 0)),
        out_shape=jax.ShapeDtypeStruct((N, D), jnp.float32),
    )(x, W_self, b2d)


def _tc_final_body(d_ref, p0_ref, p1_ref, wn_ref, o_ref):
    dn = (((1,), (1,)), ((), ()))
    o_ref[...] = d_ref[...] + lax.dot_general(
        p0_ref[...] + p1_ref[...], wn_ref[...], dn,
        preferred_element_type=jnp.float32)


def _tc_final(dense, partial, W_neigh):
    return pl.pallas_call(
        _tc_final_body,
        grid=(_NBLK,),
        in_specs=[
            pl.BlockSpec((BM, D), lambda i: (i, 0)),
            pl.BlockSpec((BM, D), lambda i: (i, 0)),
            pl.BlockSpec((BM, D), lambda i: (i + _NBLK, 0)),
            pl.BlockSpec((D, D), lambda i: (0, 0)),
        ],
        out_specs=pl.BlockSpec((BM, D), lambda i: (i, 0)),
        out_shape=jax.ShapeDtypeStruct((N, D), jnp.float32),
    )(dense, partial, partial, W_neigh)


def kernel(x, edge_index, edge_weight, W_self, b_self, W_neigh):
    ei1d = edge_index.astype(jnp.int32).reshape(2 * E)
    w1d = edge_weight.astype(jnp.float32)
    partial = _sc_aggregate(x, ei1d, w1d)
    dense = _tc_dense(x, W_self, b_self.reshape(1, D))
    return _tc_final(dense, partial, W_neigh)
